# Initial kernel scaffold; baseline (speedup 1.0000x reference)
import jax
import jax.numpy as jnp
from jax.experimental import pallas as pl


def kernel(xyz1, xyz2, points1, points2, W0, b0, g0, beta0, W1, b1, g1, beta1):
    raise NotImplementedError("write your pallas kernel here")


# speedup vs baseline: 23.0053x; 23.0053x over previous
import functools

import jax
import jax.numpy as jnp
import numpy as np
from jax import lax
from jax.experimental import pallas as pl
from jax.experimental.pallas import tpu as pltpu
from jax.experimental.pallas import tpu_sc as plsc

B, N, S, D = 8, 4096, 1024, 128
TN = 512
NT = N // TN
TNC = 512
NTC = N // TNC
INV = np.float32(1.0) / np.sqrt(np.float32(1.0 + 1e-5))

NC, NS = 2, 16
NW = NC * NS
ROWS = B * N * 3
PER_W = ROWS // NW
CH = 256
NCHUNK = PER_W // CH


def _knn_body(x1_ref, x2_ref, idx_ref, w_ref):
    b = pl.program_id(0)
    x1 = x1_ref[0]
    x2 = x2_ref[0]
    cross = lax.dot_general(x1, x2, (((1,), (1,)), ((), ())),
                            preferred_element_type=jnp.float32)
    s1 = jnp.sum(x1 * x1, axis=1, keepdims=True)
    s2 = jnp.sum(x2 * x2, axis=1)[None, :]
    d = (-2.0 * cross + s1) + s2
    iota = lax.broadcasted_iota(jnp.int32, (TN, S), 1)
    vals, idxs = [], []
    for k in range(3):
        m = jnp.min(d, axis=1, keepdims=True)
        eq = d == m
        a = jnp.min(jnp.where(eq, iota, S), axis=1, keepdims=True)
        vals.append(m)
        idxs.append(a)
        if k < 2:
            d = jnp.where(iota == a, jnp.inf, d)
    r = [1.0 / (v + 1e-8) for v in vals]
    norm = (r[0] + r[1]) + r[2]
    w_ref[0] = jnp.concatenate([r[0] / norm, r[1] / norm, r[2] / norm], axis=1)
    idx_ref[0] = jnp.concatenate(idxs, axis=1) + b * S


_KNN_GRID = (B, NT)
_KNN_IN_SPECS = [
    pl.BlockSpec((1, TN, 3), lambda b, nt: (b, nt, 0)),
    pl.BlockSpec((1, S, 3), lambda b, nt: (b, 0, 0)),
]
_KNN_OUT_SPECS = [
    pl.BlockSpec((1, TN, 3), lambda b, nt: (b, nt, 0)),
    pl.BlockSpec((1, TN, 3), lambda b, nt: (b, nt, 0)),
]
_KNN_OUT_SHAPE = [
    jax.ShapeDtypeStruct((B, N, 3), jnp.int32),
    jax.ShapeDtypeStruct((B, N, 3), jnp.float32),
]


def _knn_call(x1t, x2t):
    return pl.pallas_call(
        _knn_body,
        grid=_KNN_GRID,
        in_specs=_KNN_IN_SPECS,
        out_specs=_KNN_OUT_SPECS,
        out_shape=_KNN_OUT_SHAPE,
    )(x1t, x2t)


def _gather_body(p2_hbm, idx_hbm, out_hbm, idx_v, rows_v, sem):
    c = lax.axis_index("c")
    s = lax.axis_index("s")
    wid = s * NC + c
    base = wid * PER_W
    for j in range(NCHUNK):
        off = base + j * CH
        pltpu.sync_copy(idx_hbm.at[pl.ds(off, CH)], idx_v)
        pltpu.async_copy(p2_hbm.at[idx_v], rows_v, sem).wait()
        pltpu.sync_copy(rows_v, out_hbm.at[pl.ds(off, CH)])


def _gather_call(p2flat, idx_flat):
    k = functools.partial(
        pl.kernel,
        mesh=plsc.VectorSubcoreMesh(core_axis_name="c", subcore_axis_name="s"),
        out_type=jax.ShapeDtypeStruct((ROWS, D), jnp.float32),
        scratch_types=[
            pltpu.VMEM((CH,), jnp.int32),
            pltpu.VMEM((CH, D), jnp.float32),
            pltpu.SemaphoreType.DMA,
        ],
    )(_gather_body)
    return k(p2flat, idx_flat)


def _mlp_body(p1_ref, g_ref, w_ref, w0_ref, w1_ref, b0_ref, g0_ref,
              beta0_ref, b1_ref, g1_ref, beta1_ref, out_ref, acc_ref):
    nt = pl.program_id(1)
    p1 = p1_ref[0]
    g = g_ref[0]
    w = w_ref[0]
    interp = (g[:, 0:D] * w[:, 0:1]
              + g[:, D:2 * D] * w[:, 1:2]
              + g[:, 2 * D:3 * D] * w[:, 2:3])
    w0b = w0_ref[pl.ds(nt * TNC, TNC), :]
    low = lax.dot_general(p1, w0b, (((1,), (0,)), ((), ())),
                          preferred_element_type=jnp.float32)
    high = lax.dot_general(interp, w0b, (((0,), (0,)), ((), ())),
                           preferred_element_type=jnp.float32)
    st = jnp.concatenate([low, high], axis=0)

    @pl.when(nt == 0)
    def _():
        acc_ref[...] = st

    @pl.when(nt > 0)
    def _():
        acc_ref[...] += st

    @pl.when(nt == NTC - 1)
    def _():
        h = acc_ref[...] + b0_ref[...]
        h = jnp.maximum(h * INV * g0_ref[...] + beta0_ref[...], 0.0)
        h2 = lax.dot_general(h, w1_ref[...], (((1,), (0,)), ((), ())),
                             preferred_element_type=jnp.float32) + b1_ref[...]
        out_ref[0] = jnp.maximum(h2 * INV * g1_ref[...] + beta1_ref[...], 0.0)


_MLP_GRID = (B, NTC)
_MLP_IN_SPECS = [
    pl.BlockSpec((1, D, TNC), lambda b, nt: (b, 0, nt)),
    pl.BlockSpec((1, TNC, 3 * D), lambda b, nt: (b, nt, 0)),
    pl.BlockSpec((1, TNC, 3), lambda b, nt: (b, nt, 0)),
    pl.BlockSpec((N, 2 * D), lambda b, nt: (0, 0)),
    pl.BlockSpec((2 * D, D), lambda b, nt: (0, 0)),
    pl.BlockSpec((1, 2 * D), lambda b, nt: (0, 0)),
    pl.BlockSpec((1, 2 * D), lambda b, nt: (0, 0)),
    pl.BlockSpec((1, 2 * D), lambda b, nt: (0, 0)),
    pl.BlockSpec((1, D), lambda b, nt: (0, 0)),
    pl.BlockSpec((1, D), lambda b, nt: (0, 0)),
    pl.BlockSpec((1, D), lambda b, nt: (0, 0)),
]
_MLP_OUT_SPECS = pl.BlockSpec((1, 2 * D, D), lambda b, nt: (b, 0, 0))
_MLP_OUT_SHAPE = jax.ShapeDtypeStruct((B, 2 * D, D), jnp.float32)
_MLP_SCRATCH = [pltpu.VMEM((2 * D, 2 * D), jnp.float32)]


def _mlp_call(points1, gath, w, W0, W1, b0, g0, beta0, b1, g1, beta1):
    return pl.pallas_call(
        _mlp_body,
        grid=_MLP_GRID,
        in_specs=_MLP_IN_SPECS,
        out_specs=_MLP_OUT_SPECS,
        out_shape=_MLP_OUT_SHAPE,
        scratch_shapes=_MLP_SCRATCH,
    )(points1, gath, w, W0, W1,
      b0.reshape(1, 2 * D), g0.reshape(1, 2 * D), beta0.reshape(1, 2 * D),
      b1.reshape(1, D), g1.reshape(1, D), beta1.reshape(1, D))


def kernel(xyz1, xyz2, points1, points2, W0, b0, g0, beta0, W1, b1, g1, beta1):
    x1t = jnp.transpose(xyz1, (0, 2, 1))
    x2t = jnp.transpose(xyz2, (0, 2, 1))
    p2flat = jnp.transpose(points2, (0, 2, 1)).reshape(B * S, D)
    idxg, w = _knn_call(x1t, x2t)
    gath = _gather_call(p2flat, idxg.reshape(ROWS))
    gath = gath.reshape(B, N, 3 * D)
    return _mlp_call(points1, gath, w, W0, W1, b0, g0, beta0, b1, g1, beta1)


# speedup vs baseline: 24.8397x; 1.0797x over previous
import functools

import jax
import jax.numpy as jnp
import numpy as np
from jax import lax
from jax.experimental import pallas as pl
from jax.experimental.pallas import tpu as pltpu
from jax.experimental.pallas import tpu_sc as plsc

B, N, S, D = 8, 4096, 1024, 128
TN = 512
NT = N // TN
TNC = 512
NTC = N // TNC
INV = np.float32(1.0) / np.sqrt(np.float32(1.0 + 1e-5))

NC, NS = 2, 16
NW = NC * NS
ROWS = B * N * 3
PER_W = ROWS // NW
CH = 256
NCHUNK = PER_W // CH


def _knn_body(x1_ref, x2_ref, idx_ref, w_ref):
    b = pl.program_id(0)
    x1 = x1_ref[0]
    x2 = x2_ref[0]
    s1 = jnp.sum(x1 * x1, axis=1, keepdims=True)
    s2 = jnp.sum(x2 * x2, axis=1, keepdims=True)
    lhs = jnp.concatenate([x1, s1, jnp.ones((TN, 1), jnp.float32)], axis=1)
    rhs = jnp.concatenate([-2.0 * x2, jnp.ones((S, 1), jnp.float32), s2],
                          axis=1)
    d = lax.dot_general(lhs, rhs, (((1,), (1,)), ((), ())),
                        preferred_element_type=jnp.float32)
    iota = lax.broadcasted_iota(jnp.int32, (TN, S), 1).astype(jnp.float32)
    big = jnp.float32(S)
    vals, idxs = [], []
    for k in range(3):
        m = jnp.min(d, axis=1, keepdims=True)
        eq = d == m
        a = jnp.min(jnp.where(eq, iota, big), axis=1, keepdims=True)
        vals.append(m)
        idxs.append(a)
        if k < 2:
            d = jnp.where(iota == a, jnp.inf, d)
    r = [1.0 / (v + 1e-8) for v in vals]
    norm = (r[0] + r[1]) + r[2]
    w_ref[0] = jnp.concatenate([r[0] / norm, r[1] / norm, r[2] / norm], axis=1)
    ii = [a.astype(jnp.int32) for a in idxs]
    idx_ref[0] = jnp.concatenate(ii, axis=1) + b * S


_KNN_GRID = (B, NT)
_KNN_IN_SPECS = [
    pl.BlockSpec((1, TN, 3), lambda b, nt: (b, nt, 0)),
    pl.BlockSpec((1, S, 3), lambda b, nt: (b, 0, 0)),
]
_KNN_OUT_SPECS = [
    pl.BlockSpec((1, TN, 3), lambda b, nt: (b, nt, 0)),
    pl.BlockSpec((1, TN, 3), lambda b, nt: (b, nt, 0)),
]
_KNN_OUT_SHAPE = [
    jax.ShapeDtypeStruct((B, N, 3), jnp.int32),
    jax.ShapeDtypeStruct((B, N, 3), jnp.float32),
]


def _knn_call(x1t, x2t):
    return pl.pallas_call(
        _knn_body,
        grid=_KNN_GRID,
        in_specs=_KNN_IN_SPECS,
        out_specs=_KNN_OUT_SPECS,
        out_shape=_KNN_OUT_SHAPE,
    )(x1t, x2t)


def _gather_body(p2_hbm, idx_hbm, out_hbm, idx_v, rows_v, sem):
    c = lax.axis_index("c")
    s = lax.axis_index("s")
    wid = s * NC + c
    base = wid * PER_W
    for j in range(NCHUNK):
        off = base + j * CH
        pltpu.sync_copy(idx_hbm.at[pl.ds(off, CH)], idx_v)
        pltpu.async_copy(p2_hbm.at[idx_v], rows_v, sem).wait()
        pltpu.sync_copy(rows_v, out_hbm.at[pl.ds(off, CH)])


def _gather_call(p2flat, idx_flat):
    k = functools.partial(
        pl.kernel,
        mesh=plsc.VectorSubcoreMesh(core_axis_name="c", subcore_axis_name="s"),
        out_type=jax.ShapeDtypeStruct((ROWS, D), jnp.float32),
        scratch_types=[
            pltpu.VMEM((CH,), jnp.int32),
            pltpu.VMEM((CH, D), jnp.float32),
            pltpu.SemaphoreType.DMA,
        ],
    )(_gather_body)
    return k(p2flat, idx_flat)


def _mlp_body(p1_ref, g_ref, w_ref, w0_ref, w1_ref, b0_ref, g0_ref,
              beta0_ref, b1_ref, g1_ref, beta1_ref, out_ref, acc_ref):
    nt = pl.program_id(1)
    p1 = p1_ref[0]
    g = g_ref[0]
    w = w_ref[0]
    interp = (g[:, 0:D] * w[:, 0:1]
              + g[:, D:2 * D] * w[:, 1:2]
              + g[:, 2 * D:3 * D] * w[:, 2:3])
    w0b = w0_ref[pl.ds(nt * TNC, TNC), :]
    low = lax.dot_general(p1, w0b, (((1,), (0,)), ((), ())),
                          preferred_element_type=jnp.float32)
    high = lax.dot_general(interp, w0b, (((0,), (0,)), ((), ())),
                           preferred_element_type=jnp.float32)
    st = jnp.concatenate([low, high], axis=0)

    @pl.when(nt == 0)
    def _():
        acc_ref[...] = st

    @pl.when(nt > 0)
    def _():
        acc_ref[...] += st

    @pl.when(nt == NTC - 1)
    def _():
        h = acc_ref[...] + b0_ref[...]
        h = jnp.maximum(h * INV * g0_ref[...] + beta0_ref[...], 0.0)
        h2 = lax.dot_general(h, w1_ref[...], (((1,), (0,)), ((), ())),
                             preferred_element_type=jnp.float32) + b1_ref[...]
        out_ref[0] = jnp.maximum(h2 * INV * g1_ref[...] + beta1_ref[...], 0.0)


_MLP_GRID = (B, NTC)
_MLP_IN_SPECS = [
    pl.BlockSpec((1, D, TNC), lambda b, nt: (b, 0, nt)),
    pl.BlockSpec((1, TNC, 3 * D), lambda b, nt: (b, nt, 0)),
    pl.BlockSpec((1, TNC, 3), lambda b, nt: (b, nt, 0)),
    pl.BlockSpec((N, 2 * D), lambda b, nt: (0, 0)),
    pl.BlockSpec((2 * D, D), lambda b, nt: (0, 0)),
    pl.BlockSpec((1, 2 * D), lambda b, nt: (0, 0)),
    pl.BlockSpec((1, 2 * D), lambda b, nt: (0, 0)),
    pl.BlockSpec((1, 2 * D), lambda b, nt: (0, 0)),
    pl.BlockSpec((1, D), lambda b, nt: (0, 0)),
    pl.BlockSpec((1, D), lambda b, nt: (0, 0)),
    pl.BlockSpec((1, D), lambda b, nt: (0, 0)),
]
_MLP_OUT_SPECS = pl.BlockSpec((1, 2 * D, D), lambda b, nt: (b, 0, 0))
_MLP_OUT_SHAPE = jax.ShapeDtypeStruct((B, 2 * D, D), jnp.float32)
_MLP_SCRATCH = [pltpu.VMEM((2 * D, 2 * D), jnp.float32)]


def _mlp_call(points1, gath, w, W0, W1, b0, g0, beta0, b1, g1, beta1):
    return pl.pallas_call(
        _mlp_body,
        grid=_MLP_GRID,
        in_specs=_MLP_IN_SPECS,
        out_specs=_MLP_OUT_SPECS,
        out_shape=_MLP_OUT_SHAPE,
        scratch_shapes=_MLP_SCRATCH,
    )(points1, gath, w, W0, W1,
      b0.reshape(1, 2 * D), g0.reshape(1, 2 * D), beta0.reshape(1, 2 * D),
      b1.reshape(1, D), g1.reshape(1, D), beta1.reshape(1, D))


def kernel(xyz1, xyz2, points1, points2, W0, b0, g0, beta0, W1, b1, g1, beta1):
    x1t = jnp.transpose(xyz1, (0, 2, 1))
    x2t = jnp.transpose(xyz2, (0, 2, 1))
    p2flat = jnp.transpose(points2, (0, 2, 1)).reshape(B * S, D)
    idxg, w = _knn_call(x1t, x2t)
    gath = _gather_call(p2flat, idxg.reshape(ROWS))
    gath = gath.reshape(B, N, 3 * D)
    return _mlp_call(points1, gath, w, W0, W1, b0, g0, beta0, b1, g1, beta1)
